# baseline (speedup 1.0000x reference)
import functools

import jax
import jax.numpy as jnp
from jax import lax
from jax.experimental import pallas as pl
from jax.experimental.pallas import tpu as pltpu
from jax.experimental.pallas import tpu_sc as plsc

H, C = 4, 64
BR = 1000
NC, NS = 2, 16
G = 256
B = 128


def _elu(x):
    return jnp.where(x > 0, x, jnp.exp(jnp.minimum(x, 0.0)) - 1.0)


def _hs_scores_body(x_ref, w_ref, wa_ref, h_ref, a_ref):
    x = x_ref[...]
    h_ref[...] = jnp.dot(x, w_ref[...], preferred_element_type=jnp.float32)
    a_ref[...] = jnp.dot(x, wa_ref[...], preferred_element_type=jnp.float32)


def _hs_and_scores(x, w_src, wa_src):
    n, k = x.shape
    m = w_src.shape[1]
    return pl.pallas_call(
        _hs_scores_body,
        grid=(n // BR,),
        in_specs=[
            pl.BlockSpec((BR, k), lambda i: (i, 0)),
            pl.BlockSpec((k, m), lambda i: (0, 0)),
            pl.BlockSpec((k, H), lambda i: (0, 0)),
        ],
        out_specs=[
            pl.BlockSpec((BR, m), lambda i: (i, 0)),
            pl.BlockSpec((BR, H), lambda i: (i, 0)),
        ],
        out_shape=[
            jax.ShapeDtypeStruct((n, m), jnp.float32),
            jax.ShapeDtypeStruct((n, H), jnp.float32),
        ],
    )(x, w_src, wa_src)


def _mm_body(x_ref, w_ref, o_ref):
    o_ref[...] = jnp.dot(x_ref[...], w_ref[...], preferred_element_type=jnp.float32)


def _pallas_mm(x, w):
    n, k = x.shape
    m = w.shape[1]
    return pl.pallas_call(
        _mm_body,
        grid=(n // BR,),
        in_specs=[
            pl.BlockSpec((BR, k), lambda i: (i, 0)),
            pl.BlockSpec((k, m), lambda i: (0, 0)),
        ],
        out_specs=pl.BlockSpec((BR, m), lambda i: (i, 0)),
        out_shape=jax.ShapeDtypeStruct((n, m), jnp.float32),
    )(x, w)


def _wa(w, a):
    return jnp.einsum("khc,hc->kh", w.reshape(w.shape[0], H, C), a)


NSLAB = 25088
SL16 = NSLAB // NS
SCORE = 60000
CP = 4
EPAD = ((200000 + NS * B - 1) // (NS * B)) * (NS * B)


def _make_layer_kernel(convs, e_pad):
    ew = e_pad // NS
    nch = ew // B
    mesh = plsc.VectorSubcoreMesh(core_axis_name="c", subcore_axis_name="s")
    ncv = len(convs)

    def body(*refs):
        ins = refs[:5 * ncv]
        outs = refs[5 * ncv:7 * ncv]
        (score_v, sidx_v, didx_v, ex_all, gidx_v, gidx2_v, dloc_v, msg_v,
         msg2_v, zb_v, zf_v, acc_sp, den_sp, sem) = refs[7 * ncv:]
        c = lax.axis_index("c")
        s = lax.axis_index("s")
        z16 = jnp.zeros((16,), jnp.float32)
        for r in range(B):
            zb_v[r, pl.ds(0, 16)] = z16
        for i in range(SL16 // 16):
            zf_v[pl.ds(i * 16, 16)] = z16

        for ki in range(ncv):
            hsv, ascT, adcT, srcp, dstp = ins[5 * ki:5 * ki + 5]
            acc_out, den_out = outs[2 * ki:2 * ki + 2]
            n_src, n_dst, half, hp, sf = convs[ki]
            hm = 1 << (half.bit_length() - 1)
            pltpu.sync_copy(srcp.at[pl.ds(s * ew, ew)], sidx_v)
            pltpu.sync_copy(dstp.at[pl.ds(s * ew, ew)], didx_v)
            dbase = c * half

            def head_body(h, _, hsv=hsv, ascT=ascT, adcT=adcT,
                          acc_out=acc_out, den_out=den_out, n_src=n_src,
                          n_dst=n_dst, half=half, hp=hp, sf=sf,
                          dbase=dbase, hm=hm):
                pltpu.sync_copy(ascT.at[h], score_v.at[pl.ds(0, n_src)])
                if sf:
                    pltpu.sync_copy(adcT.at[h],
                                    score_v.at[pl.ds(n_src, n_dst)])
                else:
                    pltpu.sync_copy(adcT.at[h, pl.ds(dbase, half)],
                                    score_v.at[pl.ds(n_src, half)])
                pltpu.sync_copy(zf_v, den_sp.at[pl.ds(s * SL16, SL16)])
                plsc.subcore_barrier()

                def p_a(ci, _):
                    for j in range(B // 16):
                        off = ci * B + j * 16
                        sv = sidx_v[pl.ds(off, 16)]
                        dv = didx_v[pl.ds(off, 16)]
                        dl = dv - dbase
                        okm = (dl >= 0) & (dl < half)
                        gi = dv if sf else jnp.where(okm, dl, 0)
                        a = plsc.load_gather(score_v, [sv])
                        b = plsc.load_gather(score_v, [gi + n_src])
                        al = a + b
                        al = jnp.where(al > 0, al, al * 0.2)
                        ex_all[ci, pl.ds(j * 16, 16)] = jnp.where(
                            okm, jnp.exp(al), 0.0)
                        dloc_v[pl.ds(j * 16, 16)] = jnp.where(
                            okm, dl, dv & (hm - 1))
                    pltpu.sync_copy(ex_all.at[ci], den_sp.at[dloc_v], add=True)
                    return 0

                lax.fori_loop(0, nch, p_a, 0)
                plsc.subcore_barrier()

                @pl.when(s == 0)
                def _():
                    pltpu.sync_copy(den_sp.at[pl.ds(0, hp)], den_out.at[h, c])

                def cp_body(cp, _):
                    for k2 in range(SL16 // B):
                        pltpu.sync_copy(
                            zb_v, acc_sp.at[pl.ds(s * SL16 + k2 * B, B)])
                    rem = SL16 - (SL16 // B) * B
                    if rem:
                        pltpu.sync_copy(
                            zb_v.at[pl.ds(0, rem)],
                            acc_sp.at[pl.ds(s * SL16 + (SL16 // B) * B, rem)])
                    plsc.subcore_barrier()

                    def p_b(ci, _):
                        for j in range(B // 16):
                            off = ci * B + j * 16
                            sv = sidx_v[pl.ds(off, 16)]
                            dv = didx_v[pl.ds(off, 16)]
                            gidx_v[pl.ds(j * 16, 16)] = sv * 16 + h * CP + cp
                            dl = dv - dbase
                            okm = (dl >= 0) & (dl < half)
                            dloc_v[pl.ds(j * 16, 16)] = jnp.where(
                                okm, dl, dv & (hm - 1))
                        pltpu.async_copy(hsv.at[gidx_v], msg_v, sem).wait()
                        for g in range(B // 16):
                            exg = ex_all[ci, pl.ds(g * 16, 16)]
                            for r in range(16):
                                w = exg.at[jnp.full((16,), r, jnp.int32)].get(
                                    mode="promise_in_bounds")
                                row = g * 16 + r
                                msg_v[row, pl.ds(0, 16)] = (
                                    msg_v[row, pl.ds(0, 16)] * w)
                        pltpu.sync_copy(msg_v, acc_sp.at[dloc_v], add=True)
                        return 0

                    lax.fori_loop(0, nch, p_b, 0)
                    plsc.subcore_barrier()

                    @pl.when(s == 0)
                    def _():
                        pltpu.sync_copy(acc_sp.at[pl.ds(0, hp)],
                                        acc_out.at[h, cp, c])

                    plsc.subcore_barrier()
                    return 0

                lax.fori_loop(0, CP, cp_body, 0)
                return 0

            lax.fori_loop(0, H, head_body, 0)

    return pl.kernel(
        body,
        out_type=[
            t for (_, _, _, hp, _) in convs for t in (
                jax.ShapeDtypeStruct((H, CP, NC, hp, 16), jnp.float32),
                jax.ShapeDtypeStruct((H, NC, hp), jnp.float32),
            )
        ],
        mesh=mesh,
        compiler_params=pltpu.CompilerParams(needs_layout_passes=False,
                                             use_tc_tiling_on_sc=False),
        scratch_types=[
            pltpu.VMEM((SCORE,), jnp.float32),
            pltpu.VMEM((ew,), jnp.int32),
            pltpu.VMEM((ew,), jnp.int32),
            pltpu.VMEM((nch, B), jnp.float32),
            pltpu.VMEM((B,), jnp.int32),
            pltpu.VMEM((B,), jnp.int32),
            pltpu.VMEM((B,), jnp.int32),
            pltpu.VMEM((B, 16), jnp.float32),
            pltpu.VMEM((B, 16), jnp.float32),
            pltpu.VMEM((B, 16), jnp.float32),
            pltpu.VMEM((SL16,), jnp.float32),
            pltpu.VMEM_SHARED((NSLAB, 16), jnp.float32),
            pltpu.VMEM_SHARED((NSLAB,), jnp.float32),
            pltpu.SemaphoreType.DMA,
        ],
    )


_layer_kernels = {}


def _layer_kernel(convs, e_pad):
    key = (convs, e_pad)
    if key not in _layer_kernels:
        _layer_kernels[key] = _make_layer_kernel(convs, e_pad)
    return _layer_kernels[key]


def _conv_static(n_src, n_dst):
    sf = n_dst <= 5000
    half = n_dst // 2
    hp = ((half + 7) // 8) * 8
    return (n_src, n_dst, half, hp, sf)


def _pad_edges(src, dst, n_src, n_dst):
    e = src.shape[0]
    pad = EPAD - e
    if pad:
        ar = jnp.arange(pad, dtype=jnp.int32)
        src = jnp.concatenate([src, ar % n_src])
        dst = jnp.concatenate([dst, n_dst + (ar % G)])
    return src, dst


def _prep_conv(x_s, x_d, p):
    wa_s = _wa(p["W_src"], p["a_src"])
    wa_d = _wa(p["W_dst"], p["a_dst"])
    hs, asrc = _hs_and_scores(x_s, p["W_src"], wa_s)
    adst = _pallas_mm(x_d, wa_d)
    return (hs.reshape(-1, 16), asrc.T.copy(), adst.T.copy())


def _reassemble(acc, den, half, hp):
    a = jnp.transpose(acc, (2, 3, 0, 1, 4)).reshape(NC, hp, H * CP * 16)
    a = a[:, :half].reshape(NC * half, H * CP * 16)
    d = jnp.transpose(den, (1, 2, 0))[:, :half].reshape(NC * half, H)
    return a, d


def _run_layer(conv_inputs, edge_lists):
    statics = tuple(ci[3] for ci in conv_inputs)
    kern = _layer_kernel(statics, EPAD)
    args = []
    for (hsv, ascT, adcT, _), (srcp, dstp) in zip(conv_inputs, edge_lists):
        args += [hsv, ascT, adcT, srcp, dstp]
    outs = kern(*args)
    res = []
    for i, (_, _, half, hp, _) in enumerate(statics):
        res.append(_reassemble(outs[2 * i], outs[2 * i + 1], half, hp))
    return res


def _epi_convsum_body(a0_ref, d0_ref, a1_ref, d1_ref, b_ref, o_ref, *,
                      concat_out, act):
    parts = []
    for h in range(H):
        s0 = a0_ref[:, h * C:(h + 1) * C] / (d0_ref[:, h:h + 1] + 1e-16)
        s1 = a1_ref[:, h * C:(h + 1) * C] / (d1_ref[:, h:h + 1] + 1e-16)
        parts.append(s0 + s1)
    if concat_out:
        o = jnp.concatenate(parts, axis=1) + b_ref[...]
    else:
        o = (parts[0] + parts[1] + parts[2] + parts[3]) * 0.25 + b_ref[...]
    o_ref[...] = _elu(o) if act else o


def _epi_convsum(a0, d0, a1, d1, bias, concat_out, act):
    n = a0.shape[0]
    co = H * C if concat_out else C
    return pl.pallas_call(
        functools.partial(_epi_convsum_body, concat_out=concat_out, act=act),
        grid=(n // BR,),
        in_specs=[
            pl.BlockSpec((BR, H * C), lambda i: (i, 0)),
            pl.BlockSpec((BR, H), lambda i: (i, 0)),
            pl.BlockSpec((BR, H * C), lambda i: (i, 0)),
            pl.BlockSpec((BR, H), lambda i: (i, 0)),
            pl.BlockSpec((1, co), lambda i: (0, 0)),
        ],
        out_specs=pl.BlockSpec((BR, co), lambda i: (i, 0)),
        out_shape=jax.ShapeDtypeStruct((n, co), jnp.float32),
    )(a0, d0, a1, d1, bias.reshape(1, co))


def _epi_one_body(a_ref, d_ref, b_ref, o_ref, *, act):
    parts = []
    for h in range(H):
        parts.append(a_ref[:, h * C:(h + 1) * C] / (d_ref[:, h:h + 1] + 1e-16))
    o = jnp.concatenate(parts, axis=1) + b_ref[...]
    o_ref[...] = _elu(o) if act else o


def _epi_one(a, d, bias, act):
    n = a.shape[0]
    co = H * C
    return pl.pallas_call(
        functools.partial(_epi_one_body, act=act),
        grid=(n // BR,),
        in_specs=[
            pl.BlockSpec((BR, co), lambda i: (i, 0)),
            pl.BlockSpec((BR, H), lambda i: (i, 0)),
            pl.BlockSpec((1, co), lambda i: (0, 0)),
        ],
        out_specs=pl.BlockSpec((BR, co), lambda i: (i, 0)),
        out_shape=jax.ShapeDtypeStruct((n, co), jnp.float32),
    )(a, d, bias.reshape(1, co))


def _head_body(raw_ref, ht_ref, eps_ref, we1_ref, be1_ref, wmu_ref, bmu_ref,
               wlv_ref, blv_ref, wd1_ref, bd1_ref, wd2_ref, bd2_ref,
               w1h_ref, w1r_ref, b1_ref, w2_ref, b2_ref, w3_ref, b3_ref,
               bnsc_ref,
               logit_ref, fraud_ref, xrec_ref, mu_ref, lv_ref, rerr_ref):
    raw = raw_ref[...]
    he = jax.nn.relu(jnp.dot(raw, we1_ref[...], preferred_element_type=jnp.float32) + be1_ref[...])
    mu = jnp.dot(he, wmu_ref[...], preferred_element_type=jnp.float32) + bmu_ref[...]
    lv = jnp.dot(he, wlv_ref[...], preferred_element_type=jnp.float32) + blv_ref[...]
    z = mu + jnp.exp(0.5 * lv) * eps_ref[...]
    hdec = jax.nn.relu(jnp.dot(z, wd1_ref[...], preferred_element_type=jnp.float32) + bd1_ref[...])
    xrec = jnp.dot(hdec, wd2_ref[...], preferred_element_type=jnp.float32) + bd2_ref[...]
    rerr = jnp.mean((raw - xrec) ** 2, axis=1, keepdims=True)
    rnorm = rerr * bnsc_ref[0, 0] + bnsc_ref[0, 1]
    ht = ht_ref[...]
    hc = jnp.dot(ht, w1h_ref[...], preferred_element_type=jnp.float32)
    hc = hc + rnorm * w1r_ref[...] + b1_ref[...]
    hc = _elu(hc)
    hc = _elu(jnp.dot(hc, w2_ref[...], preferred_element_type=jnp.float32) + b2_ref[...])
    logit = jnp.dot(hc, w3_ref[...], preferred_element_type=jnp.float32) + b3_ref[...]
    logit_ref[...] = logit
    fraud_ref[...] = 1.0 / (1.0 + jnp.exp(-logit))
    xrec_ref[...] = xrec
    mu_ref[...] = mu
    lv_ref[...] = lv
    rerr_ref[...] = rerr


def _vae_cls_head(raw, h_t, eps, vae, cls, bn):
    n = raw.shape[0]
    scale = bn["gamma"][0] / jnp.sqrt(bn["rv"][0] + 1e-5)
    shift = bn["beta"][0] - bn["rm"][0] * scale
    bnsc = jnp.stack([scale, shift]).reshape(1, 2)
    row = lambda v: v.reshape(1, -1)
    full = lambda a: pl.BlockSpec((a.shape[0], a.shape[1]), lambda i: (0, 0))
    ins = [raw, h_t, eps,
           vae["We1"], row(vae["be1"]), vae["Wmu"], row(vae["bmu"]),
           vae["Wlv"], row(vae["blv"]), vae["Wd1"], row(vae["bd1"]),
           vae["Wd2"], row(vae["bd2"]),
           cls["W1"][:64], cls["W1"][64:65], row(cls["b1"]),
           cls["W2"], row(cls["b2"]), cls["W3"], row(cls["b3"]), bnsc]
    in_specs = [pl.BlockSpec((BR, 64), lambda i: (i, 0)),
                pl.BlockSpec((BR, 64), lambda i: (i, 0)),
                pl.BlockSpec((BR, 32), lambda i: (i, 0))] + [
        full(a) for a in ins[3:]]
    return pl.pallas_call(
        _head_body,
        grid=(n // BR,),
        in_specs=in_specs,
        out_specs=[pl.BlockSpec((BR, 1), lambda i: (i, 0)),
                   pl.BlockSpec((BR, 1), lambda i: (i, 0)),
                   pl.BlockSpec((BR, 64), lambda i: (i, 0)),
                   pl.BlockSpec((BR, 32), lambda i: (i, 0)),
                   pl.BlockSpec((BR, 32), lambda i: (i, 0)),
                   pl.BlockSpec((BR, 1), lambda i: (i, 0))],
        out_shape=[jax.ShapeDtypeStruct((n, 1), jnp.float32),
                   jax.ShapeDtypeStruct((n, 1), jnp.float32),
                   jax.ShapeDtypeStruct((n, 64), jnp.float32),
                   jax.ShapeDtypeStruct((n, 32), jnp.float32),
                   jax.ShapeDtypeStruct((n, 32), jnp.float32),
                   jax.ShapeDtypeStruct((n, 1), jnp.float32)],
    )(*ins)


def kernel(x_transaction, x_user, x_merchant, raw_txn_features, ei0_src, ei0_dst, ei1_src, ei1_dst, ei2_src, ei2_dst, ei3_src, ei3_dst, eps, params):
    n_t, n_u, n_m = x_transaction.shape[0], x_user.shape[0], x_merchant.shape[0]
    l0 = params["gat"]["l0"]
    l1 = params["gat"]["l1"]

    st_e0 = _conv_static(n_u, n_t)
    st_e1 = _conv_static(n_t, n_u)
    st_e2 = _conv_static(n_t, n_m)
    st_e3 = _conv_static(n_m, n_t)

    e0 = _pad_edges(ei0_src, ei0_dst, n_u, n_t)
    e1 = _pad_edges(ei1_src, ei1_dst, n_t, n_u)
    e2 = _pad_edges(ei2_src, ei2_dst, n_t, n_m)
    e3 = _pad_edges(ei3_src, ei3_dst, n_m, n_t)

    c0 = _prep_conv(x_user, x_transaction, l0["e0"]) + (st_e0,)
    c3 = _prep_conv(x_merchant, x_transaction, l0["e3"]) + (st_e3,)
    c1 = _prep_conv(x_transaction, x_user, l0["e1"]) + (st_e1,)
    c2 = _prep_conv(x_transaction, x_merchant, l0["e2"]) + (st_e2,)
    (a0, d0), (a3, d3), (a1, d1), (a2, d2) = _run_layer(
        [c0, c3, c1, c2], [e0, e3, e1, e2])

    xt1 = _epi_convsum(a0, d0, a3, d3, l0["e0"]["b"] + l0["e3"]["b"], True, True)
    xu1 = _epi_one(a1, d1, l0["e1"]["b"], True)
    xm1 = _epi_one(a2, d2, l0["e2"]["b"], True)

    c0b = _prep_conv(xu1, xt1, l1["e0"]) + (st_e0,)
    c3b = _prep_conv(xm1, xt1, l1["e3"]) + (st_e3,)
    (a0b, d0b), (a3b, d3b) = _run_layer([c0b, c3b], [e0, e3])
    h_t = _epi_convsum(a0b, d0b, a3b, d3b, l1["e0"]["b"] + l1["e3"]["b"],
                       False, False)

    logit2d, fraud2d, x_recon, mu, logvar, recon_err = _vae_cls_head(
        raw_txn_features, h_t, eps, params["vae"], params["cls"], params["bn"])
    return (logit2d[:, 0], fraud2d[:, 0], h_t, x_recon, mu, logvar, recon_err)


# speedup vs baseline: 1.1898x; 1.1898x over previous
import functools

import jax
import jax.numpy as jnp
from jax import lax
from jax.experimental import pallas as pl
from jax.experimental.pallas import tpu as pltpu
from jax.experimental.pallas import tpu_sc as plsc

H, C = 4, 64
BR = 1000
NC, NS = 2, 16
G = 256
B = 128


def _elu(x):
    return jnp.where(x > 0, x, jnp.exp(jnp.minimum(x, 0.0)) - 1.0)


def _hs_scores_body(x_ref, w_ref, wa_ref, h_ref, a_ref):
    x = x_ref[...]
    h_ref[...] = jnp.dot(x, w_ref[...], preferred_element_type=jnp.float32)
    a_ref[...] = jnp.dot(x, wa_ref[...], preferred_element_type=jnp.float32)


def _hs_and_scores(x, w_src, wa_src):
    n, k = x.shape
    m = w_src.shape[1]
    return pl.pallas_call(
        _hs_scores_body,
        grid=(n // BR,),
        in_specs=[
            pl.BlockSpec((BR, k), lambda i: (i, 0)),
            pl.BlockSpec((k, m), lambda i: (0, 0)),
            pl.BlockSpec((k, H), lambda i: (0, 0)),
        ],
        out_specs=[
            pl.BlockSpec((BR, m), lambda i: (i, 0)),
            pl.BlockSpec((BR, H), lambda i: (i, 0)),
        ],
        out_shape=[
            jax.ShapeDtypeStruct((n, m), jnp.float32),
            jax.ShapeDtypeStruct((n, H), jnp.float32),
        ],
    )(x, w_src, wa_src)


def _mm_body(x_ref, w_ref, o_ref):
    o_ref[...] = jnp.dot(x_ref[...], w_ref[...], preferred_element_type=jnp.float32)


def _pallas_mm(x, w):
    n, k = x.shape
    m = w.shape[1]
    return pl.pallas_call(
        _mm_body,
        grid=(n // BR,),
        in_specs=[
            pl.BlockSpec((BR, k), lambda i: (i, 0)),
            pl.BlockSpec((k, m), lambda i: (0, 0)),
        ],
        out_specs=pl.BlockSpec((BR, m), lambda i: (i, 0)),
        out_shape=jax.ShapeDtypeStruct((n, m), jnp.float32),
    )(x, w)


def _wa(w, a):
    return jnp.einsum("khc,hc->kh", w.reshape(w.shape[0], H, C), a)


NSLAB = 25088
SL16 = NSLAB // NS
SCORE = 60000
CP = 4
EPAD = ((200000 + NS * B - 1) // (NS * B)) * (NS * B)


def _make_layer_kernel(convs, e_pad):
    ew = e_pad // NS
    nch = ew // B
    mesh = plsc.VectorSubcoreMesh(core_axis_name="c", subcore_axis_name="s")
    ncv = len(convs)

    def body(*refs):
        ins = refs[:5 * ncv]
        outs = refs[5 * ncv:7 * ncv]
        (score_v, sidx_v, didx_v, ex_all, gidx_v, gidx2_v, dloc_v, msg_v,
         msg2_v, zb_v, acc_sp, sem, sem2) = refs[7 * ncv:]
        c = lax.axis_index("c")
        s = lax.axis_index("s")
        z16 = jnp.zeros((16,), jnp.float32)
        for r in range(B):
            zb_v[r, pl.ds(0, 16)] = z16
        for ki in range(ncv):
            hsv, ascT, adcT, srcp, dstp = ins[5 * ki:5 * ki + 5]
            acc_out, den_out = outs[2 * ki:2 * ki + 2]
            n_src, n_dst, half, hp, sf = convs[ki]
            hm = 1 << (half.bit_length() - 1)
            pltpu.sync_copy(srcp.at[pl.ds(s * ew, ew)], sidx_v)
            pltpu.sync_copy(dstp.at[pl.ds(s * ew, ew)], didx_v)
            dbase = c * half

            def head_body(h, _, hsv=hsv, ascT=ascT, adcT=adcT,
                          acc_out=acc_out, den_out=den_out, n_src=n_src,
                          n_dst=n_dst, half=half, hp=hp, sf=sf,
                          dbase=dbase, hm=hm):
                pltpu.sync_copy(ascT.at[h], score_v.at[pl.ds(0, n_src)])
                if sf:
                    pltpu.sync_copy(adcT.at[h],
                                    score_v.at[pl.ds(n_src, n_dst)])
                else:
                    pltpu.sync_copy(adcT.at[h, pl.ds(dbase, half)],
                                    score_v.at[pl.ds(n_src, half)])
                def p_a(ci, _):
                    for j in range(B // 16):
                        off = ci * B + j * 16
                        sv = sidx_v[pl.ds(off, 16)]
                        dv = didx_v[pl.ds(off, 16)]
                        dl = dv - dbase
                        okm = (dl >= 0) & (dl < half)
                        gi = dv if sf else jnp.where(okm, dl, 0)
                        a = plsc.load_gather(score_v, [sv])
                        b = plsc.load_gather(score_v, [gi + n_src])
                        al = a + b
                        al = jnp.where(al > 0, al, al * 0.2)
                        ex_all[ci, pl.ds(j * 16, 16)] = jnp.where(
                            okm, jnp.exp(al), 0.0)
                    return 0

                lax.fori_loop(0, nch, p_a, 0)

                def cp_body(cp, _):
                    for k2 in range(SL16 // B):
                        pltpu.sync_copy(
                            zb_v, acc_sp.at[pl.ds(s * SL16 + k2 * B, B)])
                    rem = SL16 - (SL16 // B) * B
                    if rem:
                        pltpu.sync_copy(
                            zb_v.at[pl.ds(0, rem)],
                            acc_sp.at[pl.ds(s * SL16 + (SL16 // B) * B, rem)])
                    plsc.subcore_barrier()

                    def p_b(qg, _):
                        ci0 = qg * 2
                        for ci, gb in ((ci0, gidx_v), (ci0 + 1, gidx2_v)):
                            for j in range(B // 16):
                                sv = sidx_v[pl.ds(ci * B + j * 16, 16)]
                                gb[pl.ds(j * 16, 16)] = sv * 16 + h * CP + cp
                        pltpu.async_copy(hsv.at[gidx_v], msg_v, sem)
                        pltpu.async_copy(hsv.at[gidx2_v], msg2_v, sem2)
                        pltpu.make_async_copy(hsv.at[gidx_v], msg_v, sem).wait()
                        pltpu.make_async_copy(hsv.at[gidx2_v], msg2_v,
                                              sem2).wait()
                        for ci, mb in ((ci0, msg_v), (ci0 + 1, msg2_v)):
                            for j in range(B // 16):
                                dv = didx_v[pl.ds(ci * B + j * 16, 16)]
                                dl = dv - dbase
                                okm = (dl >= 0) & (dl < half)
                                dloc_v[pl.ds(j * 16, 16)] = jnp.where(
                                    okm, dl, dv & (hm - 1))
                            for g in range(B // 16):
                                exg = ex_all[ci, pl.ds(g * 16, 16)]
                                for r in range(16):
                                    w = exg.at[jnp.full(
                                        (16,), r, jnp.int32)].get(
                                        mode="promise_in_bounds")
                                    row = g * 16 + r
                                    mb[row, pl.ds(0, 16)] = (
                                        mb[row, pl.ds(0, 16)] * w)
                            pltpu.sync_copy(mb, acc_sp.at[dloc_v], add=True)
                        return 0

                    lax.fori_loop(0, nch // 2, p_b, 0)
                    plsc.subcore_barrier()

                    @pl.when(s == 0)
                    def _():
                        pltpu.sync_copy(acc_sp.at[pl.ds(0, hp)],
                                        acc_out.at[h, cp, c])

                    plsc.subcore_barrier()
                    return 0

                lax.fori_loop(0, CP, cp_body, 0)

                for k2 in range(SL16 // B):
                    pltpu.sync_copy(
                        zb_v, acc_sp.at[pl.ds(s * SL16 + k2 * B, B)])
                remd = SL16 - (SL16 // B) * B
                if remd:
                    pltpu.sync_copy(
                        zb_v.at[pl.ds(0, remd)],
                        acc_sp.at[pl.ds(s * SL16 + (SL16 // B) * B, remd)])
                plsc.subcore_barrier()

                def p_d(ci, _):
                    for j in range(B // 16):
                        dv = didx_v[pl.ds(ci * B + j * 16, 16)]
                        dl = dv - dbase
                        okm = (dl >= 0) & (dl < half)
                        dloc_v[pl.ds(j * 16, 16)] = jnp.where(
                            okm, dl, dv & (hm - 1))
                    for g in range(B // 16):
                        exg = ex_all[ci, pl.ds(g * 16, 16)]
                        for r in range(16):
                            w = exg.at[jnp.full((16,), r, jnp.int32)].get(
                                mode="promise_in_bounds")
                            msg_v[g * 16 + r, pl.ds(0, 16)] = w
                    pltpu.sync_copy(msg_v, acc_sp.at[dloc_v], add=True)
                    return 0

                lax.fori_loop(0, nch, p_d, 0)
                plsc.subcore_barrier()

                @pl.when(s == 0)
                def _():
                    pltpu.sync_copy(acc_sp.at[pl.ds(0, hp)], den_out.at[h, c])

                plsc.subcore_barrier()
                return 0

            lax.fori_loop(0, H, head_body, 0)

    return pl.kernel(
        body,
        out_type=[
            t for (_, _, _, hp, _) in convs for t in (
                jax.ShapeDtypeStruct((H, CP, NC, hp, 16), jnp.float32),
                jax.ShapeDtypeStruct((H, NC, hp, 16), jnp.float32),
            )
        ],
        mesh=mesh,
        compiler_params=pltpu.CompilerParams(needs_layout_passes=False,
                                             use_tc_tiling_on_sc=False),
        scratch_types=[
            pltpu.VMEM((SCORE,), jnp.float32),
            pltpu.VMEM((ew,), jnp.int32),
            pltpu.VMEM((ew,), jnp.int32),
            pltpu.VMEM((nch, B), jnp.float32),
            pltpu.VMEM((B,), jnp.int32),
            pltpu.VMEM((B,), jnp.int32),
            pltpu.VMEM((B,), jnp.int32),
            pltpu.VMEM((B, 16), jnp.float32),
            pltpu.VMEM((B, 16), jnp.float32),
            pltpu.VMEM((B, 16), jnp.float32),
            pltpu.VMEM_SHARED((NSLAB, 16), jnp.float32),
            pltpu.SemaphoreType.DMA,
            pltpu.SemaphoreType.DMA,
        ],
    )


_layer_kernels = {}


def _layer_kernel(convs, e_pad):
    key = (convs, e_pad)
    if key not in _layer_kernels:
        _layer_kernels[key] = _make_layer_kernel(convs, e_pad)
    return _layer_kernels[key]


def _conv_static(n_src, n_dst):
    sf = n_dst <= 5000
    half = n_dst // 2
    hp = ((half + 7) // 8) * 8
    return (n_src, n_dst, half, hp, sf)


def _pad_edges(src, dst, n_src, n_dst):
    e = src.shape[0]
    pad = EPAD - e
    if pad:
        ar = jnp.arange(pad, dtype=jnp.int32)
        src = jnp.concatenate([src, ar % n_src])
        dst = jnp.concatenate([dst, n_dst + (ar % G)])
    return src, dst


def _prep_conv(x_s, x_d, p):
    wa_s = _wa(p["W_src"], p["a_src"])
    wa_d = _wa(p["W_dst"], p["a_dst"])
    hs, asrc = _hs_and_scores(x_s, p["W_src"], wa_s)
    adst = _pallas_mm(x_d, wa_d)
    return (hs.reshape(-1, 16), asrc.T.copy(), adst.T.copy())


def _reassemble(acc, den, half, hp):
    a = jnp.transpose(acc, (2, 3, 0, 1, 4)).reshape(NC, hp, H * CP * 16)
    a = a[:, :half].reshape(NC * half, H * CP * 16)
    d = jnp.transpose(den[..., 0], (1, 2, 0))[:, :half].reshape(NC * half, H)
    return a, d


def _run_layer(conv_inputs, edge_lists):
    statics = tuple(ci[3] for ci in conv_inputs)
    kern = _layer_kernel(statics, EPAD)
    args = []
    for (hsv, ascT, adcT, _), (srcp, dstp) in zip(conv_inputs, edge_lists):
        args += [hsv, ascT, adcT, srcp, dstp]
    outs = kern(*args)
    res = []
    for i, (_, _, half, hp, _) in enumerate(statics):
        res.append(_reassemble(outs[2 * i], outs[2 * i + 1], half, hp))
    return res


def _epi_convsum_body(a0_ref, d0_ref, a1_ref, d1_ref, b_ref, o_ref, *,
                      concat_out, act):
    parts = []
    for h in range(H):
        s0 = a0_ref[:, h * C:(h + 1) * C] / (d0_ref[:, h:h + 1] + 1e-16)
        s1 = a1_ref[:, h * C:(h + 1) * C] / (d1_ref[:, h:h + 1] + 1e-16)
        parts.append(s0 + s1)
    if concat_out:
        o = jnp.concatenate(parts, axis=1) + b_ref[...]
    else:
        o = (parts[0] + parts[1] + parts[2] + parts[3]) * 0.25 + b_ref[...]
    o_ref[...] = _elu(o) if act else o


def _epi_convsum(a0, d0, a1, d1, bias, concat_out, act):
    n = a0.shape[0]
    co = H * C if concat_out else C
    return pl.pallas_call(
        functools.partial(_epi_convsum_body, concat_out=concat_out, act=act),
        grid=(n // BR,),
        in_specs=[
            pl.BlockSpec((BR, H * C), lambda i: (i, 0)),
            pl.BlockSpec((BR, H), lambda i: (i, 0)),
            pl.BlockSpec((BR, H * C), lambda i: (i, 0)),
            pl.BlockSpec((BR, H), lambda i: (i, 0)),
            pl.BlockSpec((1, co), lambda i: (0, 0)),
        ],
        out_specs=pl.BlockSpec((BR, co), lambda i: (i, 0)),
        out_shape=jax.ShapeDtypeStruct((n, co), jnp.float32),
    )(a0, d0, a1, d1, bias.reshape(1, co))


def _epi_one_body(a_ref, d_ref, b_ref, o_ref, *, act):
    parts = []
    for h in range(H):
        parts.append(a_ref[:, h * C:(h + 1) * C] / (d_ref[:, h:h + 1] + 1e-16))
    o = jnp.concatenate(parts, axis=1) + b_ref[...]
    o_ref[...] = _elu(o) if act else o


def _epi_one(a, d, bias, act):
    n = a.shape[0]
    co = H * C
    return pl.pallas_call(
        functools.partial(_epi_one_body, act=act),
        grid=(n // BR,),
        in_specs=[
            pl.BlockSpec((BR, co), lambda i: (i, 0)),
            pl.BlockSpec((BR, H), lambda i: (i, 0)),
            pl.BlockSpec((1, co), lambda i: (0, 0)),
        ],
        out_specs=pl.BlockSpec((BR, co), lambda i: (i, 0)),
        out_shape=jax.ShapeDtypeStruct((n, co), jnp.float32),
    )(a, d, bias.reshape(1, co))


def _head_body(raw_ref, ht_ref, eps_ref, we1_ref, be1_ref, wmu_ref, bmu_ref,
               wlv_ref, blv_ref, wd1_ref, bd1_ref, wd2_ref, bd2_ref,
               w1h_ref, w1r_ref, b1_ref, w2_ref, b2_ref, w3_ref, b3_ref,
               bnsc_ref,
               logit_ref, fraud_ref, xrec_ref, mu_ref, lv_ref, rerr_ref):
    raw = raw_ref[...]
    he = jax.nn.relu(jnp.dot(raw, we1_ref[...], preferred_element_type=jnp.float32) + be1_ref[...])
    mu = jnp.dot(he, wmu_ref[...], preferred_element_type=jnp.float32) + bmu_ref[...]
    lv = jnp.dot(he, wlv_ref[...], preferred_element_type=jnp.float32) + blv_ref[...]
    z = mu + jnp.exp(0.5 * lv) * eps_ref[...]
    hdec = jax.nn.relu(jnp.dot(z, wd1_ref[...], preferred_element_type=jnp.float32) + bd1_ref[...])
    xrec = jnp.dot(hdec, wd2_ref[...], preferred_element_type=jnp.float32) + bd2_ref[...]
    rerr = jnp.mean((raw - xrec) ** 2, axis=1, keepdims=True)
    rnorm = rerr * bnsc_ref[0, 0] + bnsc_ref[0, 1]
    ht = ht_ref[...]
    hc = jnp.dot(ht, w1h_ref[...], preferred_element_type=jnp.float32)
    hc = hc + rnorm * w1r_ref[...] + b1_ref[...]
    hc = _elu(hc)
    hc = _elu(jnp.dot(hc, w2_ref[...], preferred_element_type=jnp.float32) + b2_ref[...])
    logit = jnp.dot(hc, w3_ref[...], preferred_element_type=jnp.float32) + b3_ref[...]
    logit_ref[...] = logit
    fraud_ref[...] = 1.0 / (1.0 + jnp.exp(-logit))
    xrec_ref[...] = xrec
    mu_ref[...] = mu
    lv_ref[...] = lv
    rerr_ref[...] = rerr


def _vae_cls_head(raw, h_t, eps, vae, cls, bn):
    n = raw.shape[0]
    scale = bn["gamma"][0] / jnp.sqrt(bn["rv"][0] + 1e-5)
    shift = bn["beta"][0] - bn["rm"][0] * scale
    bnsc = jnp.stack([scale, shift]).reshape(1, 2)
    row = lambda v: v.reshape(1, -1)
    full = lambda a: pl.BlockSpec((a.shape[0], a.shape[1]), lambda i: (0, 0))
    ins = [raw, h_t, eps,
           vae["We1"], row(vae["be1"]), vae["Wmu"], row(vae["bmu"]),
           vae["Wlv"], row(vae["blv"]), vae["Wd1"], row(vae["bd1"]),
           vae["Wd2"], row(vae["bd2"]),
           cls["W1"][:64], cls["W1"][64:65], row(cls["b1"]),
           cls["W2"], row(cls["b2"]), cls["W3"], row(cls["b3"]), bnsc]
    in_specs = [pl.BlockSpec((BR, 64), lambda i: (i, 0)),
                pl.BlockSpec((BR, 64), lambda i: (i, 0)),
                pl.BlockSpec((BR, 32), lambda i: (i, 0))] + [
        full(a) for a in ins[3:]]
    return pl.pallas_call(
        _head_body,
        grid=(n // BR,),
        in_specs=in_specs,
        out_specs=[pl.BlockSpec((BR, 1), lambda i: (i, 0)),
                   pl.BlockSpec((BR, 1), lambda i: (i, 0)),
                   pl.BlockSpec((BR, 64), lambda i: (i, 0)),
                   pl.BlockSpec((BR, 32), lambda i: (i, 0)),
                   pl.BlockSpec((BR, 32), lambda i: (i, 0)),
                   pl.BlockSpec((BR, 1), lambda i: (i, 0))],
        out_shape=[jax.ShapeDtypeStruct((n, 1), jnp.float32),
                   jax.ShapeDtypeStruct((n, 1), jnp.float32),
                   jax.ShapeDtypeStruct((n, 64), jnp.float32),
                   jax.ShapeDtypeStruct((n, 32), jnp.float32),
                   jax.ShapeDtypeStruct((n, 32), jnp.float32),
                   jax.ShapeDtypeStruct((n, 1), jnp.float32)],
    )(*ins)


def kernel(x_transaction, x_user, x_merchant, raw_txn_features, ei0_src, ei0_dst, ei1_src, ei1_dst, ei2_src, ei2_dst, ei3_src, ei3_dst, eps, params):
    n_t, n_u, n_m = x_transaction.shape[0], x_user.shape[0], x_merchant.shape[0]
    l0 = params["gat"]["l0"]
    l1 = params["gat"]["l1"]

    st_e0 = _conv_static(n_u, n_t)
    st_e1 = _conv_static(n_t, n_u)
    st_e2 = _conv_static(n_t, n_m)
    st_e3 = _conv_static(n_m, n_t)

    e0 = _pad_edges(ei0_src, ei0_dst, n_u, n_t)
    e1 = _pad_edges(ei1_src, ei1_dst, n_t, n_u)
    e2 = _pad_edges(ei2_src, ei2_dst, n_t, n_m)
    e3 = _pad_edges(ei3_src, ei3_dst, n_m, n_t)

    c0 = _prep_conv(x_user, x_transaction, l0["e0"]) + (st_e0,)
    c3 = _prep_conv(x_merchant, x_transaction, l0["e3"]) + (st_e3,)
    c1 = _prep_conv(x_transaction, x_user, l0["e1"]) + (st_e1,)
    c2 = _prep_conv(x_transaction, x_merchant, l0["e2"]) + (st_e2,)
    (a0, d0), (a3, d3), (a1, d1), (a2, d2) = _run_layer(
        [c0, c3, c1, c2], [e0, e3, e1, e2])

    xt1 = _epi_convsum(a0, d0, a3, d3, l0["e0"]["b"] + l0["e3"]["b"], True, True)
    xu1 = _epi_one(a1, d1, l0["e1"]["b"], True)
    xm1 = _epi_one(a2, d2, l0["e2"]["b"], True)

    c0b = _prep_conv(xu1, xt1, l1["e0"]) + (st_e0,)
    c3b = _prep_conv(xm1, xt1, l1["e3"]) + (st_e3,)
    (a0b, d0b), (a3b, d3b) = _run_layer([c0b, c3b], [e0, e3])
    h_t = _epi_convsum(a0b, d0b, a3b, d3b, l1["e0"]["b"] + l1["e3"]["b"],
                       False, False)

    logit2d, fraud2d, x_recon, mu, logvar, recon_err = _vae_cls_head(
        raw_txn_features, h_t, eps, params["vae"], params["cls"], params["bn"])
    return (logit2d[:, 0], fraud2d[:, 0], h_t, x_recon, mu, logvar, recon_err)
